# baseline (speedup 1.0000x reference)
import functools

import jax
import jax.numpy as jnp
from jax import lax
from jax.experimental import pallas as pl
from jax.experimental.pallas import tpu as pltpu
from jax.experimental.pallas import tpu_sc as plsc

_BATCH = 16384
_A = 1024
_SIZE = 64
_RANGE = 4.0
_H = 2.0 * _RANGE / (_SIZE - 1)
_INVH = 1.0 / _H

_NC = 2
_NS = 16
_NSTRIP = 8
_NGRP = 4
_AW = _A // _NSTRIP
_ROWS = _BATCH // _NGRP
_NB = 128
_NCHUNK = _ROWS // _NB
_NTAB = _AW * _SIZE
_TSTRIDE = _SIZE - 1
_NH = _AW // 16
_HALF = _NB // 2
_NRB = _HALF // 16

_mesh = plsc.VectorSubcoreMesh(core_axis_name="c", subcore_axis_name="s")


@functools.partial(
    pl.kernel,
    out_type=(
        jax.ShapeDtypeStruct((_BATCH, _A), jnp.float32),
        jax.ShapeDtypeStruct((_A, _BATCH), jnp.float32),
    ),
    mesh=_mesh,
    scratch_types=[
        pltpu.VMEM((_AW * _TSTRIDE,), jnp.float32),
        pltpu.VMEM((_AW * _TSTRIDE,), jnp.float32),
        pltpu.VMEM((_NB, _AW), jnp.float32),
        pltpu.VMEM((_NB, _AW), jnp.float32),
        pltpu.VMEM((_NB, _AW), jnp.float32),
        pltpu.VMEM((_NB, _AW), jnp.float32),
        pltpu.VMEM((_AW, _NB), jnp.float32),
        pltpu.VMEM((_AW, _NB), jnp.float32),
        pltpu.VMEM((_NRB * _NH * 272,), jnp.float32),
        pltpu.SemaphoreType.DMA,
        pltpu.SemaphoreType.DMA,
        pltpu.SemaphoreType.DMA,
    ],
    compiler_params=pltpu.CompilerParams(needs_layout_passes=False),
)
def _spline_sc(x_hbm, cv_hbm, out_hbm, grad_hbm, atab, btab, xb0, xb1,
               ob0, ob1, gb0, gb1, sbuf, sem_x, sem_o0, sem_o1):
    wid = lax.axis_index("s") * _NC + lax.axis_index("c")
    s_col = wid % _NSTRIP
    g_row = wid // _NSTRIP
    a0 = s_col * _AW
    r0 = g_row * _ROWS
    xb = (xb0, xb1)
    ob = (ob0, ob1)
    gb = (gb0, gb1)
    sem_o = (sem_o0, sem_o1)

    def xsrc(ci):
        return x_hbm.at[pl.ds(r0 + ci * _NB, _NB), pl.ds(a0, _AW)]

    def odst(ci):
        return out_hbm.at[pl.ds(r0 + ci * _NB, _NB), pl.ds(a0, _AW)]

    def gdst(ci):
        return grad_hbm.at[pl.ds(a0, _AW), pl.ds(r0 + ci * _NB, _NB)]

    pltpu.async_copy(xsrc(0), xb0, sem_x)
    cvrow = pl.multiple_of(a0 * _SIZE // _NB, 64)
    pltpu.sync_copy(cv_hbm.at[pl.ds(cvrow, _NTAB // _NB), :],
                    gb1.at[pl.ds(0, _NTAB // _NB), :])

    iota = lax.iota(jnp.int32, 16)
    bases = [iota * _TSTRIDE + 16 * h * _TSTRIDE for h in range(_NH)]
    iota17 = iota * 17

    @plsc.parallel_loop(0, _NTAB, 16, unroll=4)
    def _prep(k):
        c0 = gb1[k >> 7, pl.ds(k & 127, 16)]
        kv = iota + k
        kv1 = jnp.minimum(kv + 1, _NTAB - 1)
        c1 = plsc.load_gather(gb1, [kv1 >> 7, kv1 & 127])
        beta = (c1 - c0) * _INVH
        lane = kv & (_SIZE - 1)
        knot = lane.astype(jnp.float32) * _H - _RANGE
        dst = iota + (k - (k >> 6))
        msk = lane < (_SIZE - 1)
        plsc.store_scatter(btab, [dst], beta, mask=msk)
        plsc.store_scatter(atab, [dst], c0 - beta * knot, mask=msk)

    def pair_body(i, carry):
        for ph in range(2):
            ci = 2 * i + ph
            xb_c, ob_c, gb_c = xb[ph], ob[ph], gb[ph]
            pltpu.make_async_copy(xsrc(ci), xb_c, sem_x).wait()

            @pl.when(ci < _NCHUNK - 1)
            def _prefetch():
                pltpu.async_copy(xsrc(ci + 1), xb[1 - ph], sem_x)

            @pl.when(ci >= 2)
            def _drain():
                pltpu.make_async_copy(ob_c, odst(ci - 2), sem_o[ph]).wait()
                pltpu.make_async_copy(gb_c, gdst(ci - 2), sem_o[ph]).wait()

            for half in range(_NB // _HALF):
                hbase = half * _HALF

                @plsc.parallel_loop(0, _HALF, 1, unroll=2)
                def _p1(r):
                    rr = hbase + r
                    sboff = (r >> 4) * (_NH * 272) + (r & 15) * 17
                    for h in range(_NH):
                        v = xb_c[rr, pl.ds(16 * h, 16)]
                        t = v * _INVH + (_RANGE * _INVH)
                        t = jnp.minimum(jnp.maximum(t, 0.0), float(_SIZE - 2))
                        idx = t.astype(jnp.int32) + bases[h]
                        beta = plsc.load_gather(btab, [idx])
                        alpha = plsc.load_gather(atab, [idx])
                        ob_c[rr, pl.ds(16 * h, 16)] = alpha + beta * v
                        sbuf[pl.ds(h * 272 + sboff, 16)] = beta

                @plsc.parallel_loop(0, _HALF, 1, unroll=2)
                def _p2(j):
                    c = j & 15
                    tbase = (j >> 4) * (_NH * 272)
                    idx_c = iota17 + c
                    for h in range(_NH):
                        col = plsc.load_gather(
                            sbuf.at[pl.ds(tbase + h * 272, 272)], [idx_c])
                        gb_c[16 * h + c, pl.ds(hbase + (j >> 4) * 16, 16)] = col

            pltpu.async_copy(ob_c, odst(ci), sem_o[ph])
            pltpu.async_copy(gb_c, gdst(ci), sem_o[ph])
        return carry

    lax.fori_loop(0, _NCHUNK // 2, pair_body, 0)
    for ci in (_NCHUNK - 2, _NCHUNK - 1):
        ph = ci % 2
        pltpu.make_async_copy(ob[ph], odst(ci), sem_o[ph]).wait()
        pltpu.make_async_copy(gb[ph], gdst(ci), sem_o[ph]).wait()


def kernel(x, coefficients_vect, nodal_val_loc_tensor, zero_knot_indexes):
    del nodal_val_loc_tensor, zero_knot_indexes
    return _spline_sc(x, coefficients_vect.reshape(_A * _SIZE // _NB, _NB))


# speedup vs baseline: 1.1769x; 1.1769x over previous
import functools

import jax
import jax.numpy as jnp
from jax import lax
from jax.experimental import pallas as pl
from jax.experimental.pallas import tpu as pltpu
from jax.experimental.pallas import tpu_sc as plsc

_BATCH = 16384
_A = 1024
_SIZE = 64
_RANGE = 4.0
_H = 2.0 * _RANGE / (_SIZE - 1)
_INVH = 1.0 / _H

_NC = 2
_NS = 16
_NSTRIP = 8
_NGRP = 4
_AW = _A // _NSTRIP
_ROWS = _BATCH // _NGRP
_NB = 128
_NCHUNK = _ROWS // _NB
_NTAB = _AW * _SIZE
_TSTRIDE = _SIZE - 1

_mesh = plsc.VectorSubcoreMesh(core_axis_name="c", subcore_axis_name="s")


@functools.partial(
    pl.kernel,
    out_type=(
        jax.ShapeDtypeStruct((_BATCH, _A), jnp.float32),
        jax.ShapeDtypeStruct((_A, _BATCH), jnp.float32),
    ),
    mesh=_mesh,
    scratch_types=[
        pltpu.VMEM((_NTAB,), jnp.float32),
        pltpu.VMEM((_AW * _TSTRIDE,), jnp.float32),
        pltpu.VMEM((_AW * _TSTRIDE,), jnp.float32),
        pltpu.VMEM((_NB, _AW), jnp.float32),
        pltpu.VMEM((_NB, _AW), jnp.float32),
        pltpu.VMEM((_NB, _AW), jnp.float32),
        pltpu.VMEM((_NB, _AW), jnp.float32),
        pltpu.VMEM((_AW, _NB), jnp.float32),
        pltpu.VMEM((_AW, _NB), jnp.float32),
        pltpu.VMEM((_AW // 16 * 16 * 17,), jnp.float32),
        pltpu.SemaphoreType.DMA,
        pltpu.SemaphoreType.DMA,
        pltpu.SemaphoreType.DMA,
    ],
    compiler_params=pltpu.CompilerParams(needs_layout_passes=False),
)
def _spline_sc(x_hbm, cv_hbm, out_hbm, grad_hbm, ctab, atab, btab, xb0, xb1,
               ob0, ob1, gb0, gb1, sbuf, sem_x, sem_o0, sem_o1):
    wid = lax.axis_index("s") * _NC + lax.axis_index("c")
    s_col = wid % _NSTRIP
    g_row = wid // _NSTRIP
    a0 = s_col * _AW
    r0 = g_row * _ROWS
    xb = (xb0, xb1)
    ob = (ob0, ob1)
    gb = (gb0, gb1)
    sem_o = (sem_o0, sem_o1)

    def xsrc(ci):
        return x_hbm.at[pl.ds(r0 + ci * _NB, _NB), pl.ds(a0, _AW)]

    def odst(ci):
        return out_hbm.at[pl.ds(r0 + ci * _NB, _NB), pl.ds(a0, _AW)]

    def gdst(ci):
        return grad_hbm.at[pl.ds(a0, _AW), pl.ds(r0 + ci * _NB, _NB)]

    pltpu.async_copy(xsrc(0), xb0, sem_x)
    pltpu.sync_copy(cv_hbm.at[pl.ds(a0 * _SIZE, _NTAB)], ctab)

    iota = lax.iota(jnp.int32, 16)
    bases = [iota * _TSTRIDE + 16 * h * _TSTRIDE for h in range(_AW // 16)]
    iota17 = iota * 17

    @plsc.parallel_loop(0, _NTAB, 16, unroll=4)
    def _prep(k):
        c0 = ctab[pl.ds(k, 16)]
        kv = iota + k
        c1 = plsc.load_gather(ctab, [jnp.minimum(kv + 1, _NTAB - 1)])
        beta = (c1 - c0) * _INVH
        lane = kv & (_SIZE - 1)
        knot = lane.astype(jnp.float32) * _H - _RANGE
        dst = iota + (k - (k >> 6))
        msk = lane < (_SIZE - 1)
        plsc.store_scatter(btab, [dst], beta, mask=msk)
        plsc.store_scatter(atab, [dst], c0 - beta * knot, mask=msk)

    def pair_body(i, carry):
        for ph in range(2):
            ci = 2 * i + ph
            xb_c, ob_c, gb_c = xb[ph], ob[ph], gb[ph]
            pltpu.make_async_copy(xsrc(ci), xb_c, sem_x).wait()

            @pl.when(ci < _NCHUNK - 1)
            def _prefetch():
                pltpu.async_copy(xsrc(ci + 1), xb[1 - ph], sem_x)

            @pl.when(ci >= 2)
            def _drain():
                pltpu.make_async_copy(ob_c, odst(ci - 2), sem_o[ph]).wait()
                pltpu.make_async_copy(gb_c, gdst(ci - 2), sem_o[ph]).wait()

            def rb_body(rb, c2):
                rbase = rb * 16

                @plsc.parallel_loop(0, 16, 1, unroll=2)
                def _p1(r):
                    rr = rbase + r
                    sboff = r * 17
                    for h in range(_AW // 16):
                        v = xb_c[rr, pl.ds(16 * h, 16)]
                        t = v * _INVH + (_RANGE * _INVH)
                        t = jnp.minimum(jnp.maximum(t, 0.0), float(_SIZE - 2))
                        idx = t.astype(jnp.int32) + bases[h]
                        beta = plsc.load_gather(btab, [idx])
                        alpha = plsc.load_gather(atab, [idx])
                        ob_c[rr, pl.ds(16 * h, 16)] = alpha + beta * v
                        sbuf[pl.ds(h * 272 + sboff, 16)] = beta

                @plsc.parallel_loop(0, 16, 1, unroll=2)
                def _p2(c):
                    idx_c = iota17 + c
                    for h in range(_AW // 16):
                        col = plsc.load_gather(
                            sbuf.at[pl.ds(h * 272, 272)], [idx_c])
                        gb_c[16 * h + c, pl.ds(rbase, 16)] = col
                return c2

            lax.fori_loop(0, _NB // 16, rb_body, 0)

            pltpu.async_copy(ob_c, odst(ci), sem_o[ph])
            pltpu.async_copy(gb_c, gdst(ci), sem_o[ph])
        return carry

    lax.fori_loop(0, _NCHUNK // 2, pair_body, 0)
    for ci in (_NCHUNK - 2, _NCHUNK - 1):
        ph = ci % 2
        pltpu.make_async_copy(ob[ph], odst(ci), sem_o[ph]).wait()
        pltpu.make_async_copy(gb[ph], gdst(ci), sem_o[ph]).wait()


def kernel(x, coefficients_vect, nodal_val_loc_tensor, zero_knot_indexes):
    del nodal_val_loc_tensor, zero_knot_indexes
    return _spline_sc(x, coefficients_vect)


# speedup vs baseline: 1.2198x; 1.0365x over previous
import functools

import jax
import jax.numpy as jnp
from jax import lax
from jax.experimental import pallas as pl
from jax.experimental.pallas import tpu as pltpu
from jax.experimental.pallas import tpu_sc as plsc

_BATCH = 16384
_A = 1024
_SIZE = 64
_RANGE = 4.0
_H = 2.0 * _RANGE / (_SIZE - 1)
_INVH = 1.0 / _H

_NC = 2
_NS = 16
_NSTRIP = 8
_NGRP = 4
_AW = _A // _NSTRIP
_ROWS = _BATCH // _NGRP
_NB = 128
_NCHUNK = _ROWS // _NB
_NTAB = _AW * _SIZE
_TSTRIDE = _SIZE - 1

_mesh = plsc.VectorSubcoreMesh(core_axis_name="c", subcore_axis_name="s")


@functools.partial(
    pl.kernel,
    out_type=(
        jax.ShapeDtypeStruct((_BATCH, _A), jnp.float32),
        jax.ShapeDtypeStruct((_A, _BATCH), jnp.float32),
    ),
    mesh=_mesh,
    scratch_types=[
        pltpu.VMEM((_NTAB,), jnp.float32),
        pltpu.VMEM((_AW * _TSTRIDE,), jnp.float32),
        pltpu.VMEM((_AW * _TSTRIDE,), jnp.float32),
        pltpu.VMEM((_NB, _AW), jnp.float32),
        pltpu.VMEM((_NB, _AW), jnp.float32),
        pltpu.VMEM((_NB, _AW), jnp.float32),
        pltpu.VMEM((_NB, _AW), jnp.float32),
        pltpu.VMEM((_AW, _NB), jnp.float32),
        pltpu.VMEM((_AW, _NB), jnp.float32),
        pltpu.VMEM((_AW // 16 * 16 * 17,), jnp.float32),
        pltpu.SemaphoreType.DMA,
        pltpu.SemaphoreType.DMA,
        pltpu.SemaphoreType.DMA,
    ],
    compiler_params=pltpu.CompilerParams(needs_layout_passes=False),
)
def _spline_sc(x_hbm, cv_hbm, out_hbm, grad_hbm, ctab, atab, btab, xb0, xb1,
               ob0, ob1, gb0, gb1, sbuf, sem_x, sem_o0, sem_o1):
    wid = lax.axis_index("s") * _NC + lax.axis_index("c")
    s_col = wid % _NSTRIP
    g_row = wid // _NSTRIP
    a0 = s_col * _AW
    r0 = g_row * _ROWS
    xb = (xb0, xb1)
    ob = (ob0, ob1)
    gb = (gb0, gb1)
    sem_o = (sem_o0, sem_o1)

    def xsrc(ci):
        return x_hbm.at[pl.ds(r0 + ci * _NB, _NB), pl.ds(a0, _AW)]

    def odst(ci):
        return out_hbm.at[pl.ds(r0 + ci * _NB, _NB), pl.ds(a0, _AW)]

    def gdst(ci):
        return grad_hbm.at[pl.ds(a0, _AW), pl.ds(r0 + ci * _NB, _NB)]

    pltpu.async_copy(xsrc(0), xb0, sem_x)
    pltpu.sync_copy(cv_hbm.at[pl.ds(a0 * _SIZE, _NTAB)], ctab)

    iota = lax.iota(jnp.int32, 16)
    bases = [iota * _TSTRIDE + 16 * h * _TSTRIDE for h in range(_AW // 16)]
    iota17 = iota * 17

    @plsc.parallel_loop(0, _NTAB, 16, unroll=4)
    def _prep(k):
        c0 = ctab[pl.ds(k, 16)]
        kv = iota + k
        c1 = plsc.load_gather(ctab, [jnp.minimum(kv + 1, _NTAB - 1)])
        beta = (c1 - c0) * _INVH
        lane = kv & (_SIZE - 1)
        knot = lane.astype(jnp.float32) * _H - _RANGE
        dst = iota + (k - (k >> 6))
        msk = lane < (_SIZE - 1)
        plsc.store_scatter(btab, [dst], beta, mask=msk)
        plsc.store_scatter(atab, [dst], c0 - beta * knot, mask=msk)

    def pair_body(i, carry):
        for ph in range(2):
            ci = 2 * i + ph
            xb_c, ob_c, gb_c = xb[ph], ob[ph], gb[ph]
            pltpu.make_async_copy(xsrc(ci), xb_c, sem_x).wait()

            @pl.when(ci < _NCHUNK - 1)
            def _prefetch():
                pltpu.async_copy(xsrc(ci + 1), xb[1 - ph], sem_x)

            @pl.when(ci >= 2)
            def _drain():
                pltpu.make_async_copy(ob_c, odst(ci - 2), sem_o[ph]).wait()
                pltpu.make_async_copy(gb_c, gdst(ci - 2), sem_o[ph]).wait()

            def rb_body(rb, c2):
                rbase = rb * 16

                @plsc.parallel_loop(0, 16, 1, unroll=4)
                def _p1(r):
                    rr = rbase + r
                    sboff = r * 17
                    for h in range(_AW // 16):
                        v = xb_c[rr, pl.ds(16 * h, 16)]
                        t = v * _INVH + (_RANGE * _INVH)
                        t = jnp.minimum(jnp.maximum(t, 0.0), float(_SIZE - 2))
                        idx = t.astype(jnp.int32) + bases[h]
                        beta = plsc.load_gather(btab, [idx])
                        alpha = plsc.load_gather(atab, [idx])
                        ob_c[rr, pl.ds(16 * h, 16)] = alpha + beta * v
                        sbuf[pl.ds(h * 272 + sboff, 16)] = beta

                @plsc.parallel_loop(0, 16, 1, unroll=4)
                def _p2(c):
                    idx_c = iota17 + c
                    for h in range(_AW // 16):
                        col = plsc.load_gather(
                            sbuf.at[pl.ds(h * 272, 272)], [idx_c])
                        gb_c[16 * h + c, pl.ds(rbase, 16)] = col
                return c2

            lax.fori_loop(0, _NB // 16, rb_body, 0)

            pltpu.async_copy(ob_c, odst(ci), sem_o[ph])
            pltpu.async_copy(gb_c, gdst(ci), sem_o[ph])
        return carry

    lax.fori_loop(0, _NCHUNK // 2, pair_body, 0)
    for ci in (_NCHUNK - 2, _NCHUNK - 1):
        ph = ci % 2
        pltpu.make_async_copy(ob[ph], odst(ci), sem_o[ph]).wait()
        pltpu.make_async_copy(gb[ph], gdst(ci), sem_o[ph]).wait()


def kernel(x, coefficients_vect, nodal_val_loc_tensor, zero_knot_indexes):
    del nodal_val_loc_tensor, zero_knot_indexes
    return _spline_sc(x, coefficients_vect)
